# baseline (device time: 11120 ns/iter reference)
import jax
import jax.numpy as jnp
from jax import lax
from jax.experimental import pallas as pl
from jax.experimental.pallas import tpu as pltpu

N_DEV = 4
E_PER = 2
N_EXP = N_DEV * E_PER


def kernel(x, router_W, route_idx, expert_W):
    m, d = x.shape
    e_per, _, h = expert_W.shape

    def body(x_ref, rw_ref, idx_ref, ew_ref, out_ref,
             my_bf_ref, comm_ref, send_sems, recv_sems, local_sem):
        my = lax.axis_index("i")

        my_bf_ref[...] = ew_ref[...].astype(jnp.bfloat16)

        barrier_sem = pltpu.get_barrier_semaphore()
        for k in range(1, N_DEV):
            pl.semaphore_signal(
                barrier_sem, inc=1,
                device_id=(lax.rem(my + k, N_DEV),),
                device_id_type=pl.DeviceIdType.MESH,
            )
        pl.semaphore_wait(barrier_sem, N_DEV - 1)

        sends = []
        for k in range(1, N_DEV):
            rdma = pltpu.make_async_remote_copy(
                src_ref=my_bf_ref,
                dst_ref=comm_ref.at[my],
                send_sem=send_sems.at[k],
                recv_sem=recv_sems.at[my],
                device_id=(lax.rem(my + k, N_DEV),),
                device_id_type=pl.DeviceIdType.MESH,
            )
            rdma.start()
            sends.append(rdma)
        local_cp = pltpu.make_async_copy(my_bf_ref, comm_ref.at[my], local_sem)
        local_cp.start()

        xv = x_ref[...]
        xb = xv.astype(jnp.bfloat16)
        scores = jnp.dot(xv, rw_ref[...], preferred_element_type=jnp.float32)
        p = jnp.exp(scores - jnp.max(scores, axis=-1, keepdims=True))
        p = p / jnp.sum(p, axis=-1, keepdims=True)
        iota8 = lax.broadcasted_iota(jnp.int32, (m, N_EXP), 1)
        m0 = (iota8 == idx_ref[:, 0:1]).astype(jnp.float32)
        m1 = (iota8 == idx_ref[:, 1:2]).astype(jnp.float32)
        g0 = jnp.sum(p * m0, axis=-1, keepdims=True)
        g1 = jnp.sum(p * m1, axis=-1, keepdims=True)
        gates = (g0 * m0 + g1 * m1) / (g0 + g1)

        gb = gates.astype(jnp.bfloat16)
        xs = jnp.concatenate(
            [gb[:, e:e + 1] * xb for e in range(N_EXP)], axis=1
        )

        acc = jnp.zeros((m, h), jnp.float32)
        for o in range(N_DEV):
            @pl.when(o != my)
            def _():
                recv = pltpu.make_async_remote_copy(
                    src_ref=my_bf_ref,
                    dst_ref=comm_ref.at[o],
                    send_sem=send_sems.at[0],
                    recv_sem=recv_sems.at[o],
                    device_id=(my,),
                    device_id_type=pl.DeviceIdType.MESH,
                )
                recv.wait_recv()

            @pl.when(o == my)
            def _():
                local_cp.wait()
            w = jnp.reshape(comm_ref[o], (e_per * d, h))
            acc = acc + jnp.dot(
                xs[:, o * e_per * d:(o + 1) * e_per * d], w,
                preferred_element_type=jnp.float32,
            )
        out_ref[...] = acc

        for rdma in sends:
            rdma.wait_send()

    return pl.pallas_call(
        body,
        out_shape=jax.ShapeDtypeStruct((m, h), jnp.float32),
        in_specs=[
            pl.BlockSpec(memory_space=pltpu.VMEM),
            pl.BlockSpec(memory_space=pltpu.VMEM),
            pl.BlockSpec(memory_space=pltpu.VMEM),
            pl.BlockSpec(memory_space=pltpu.VMEM),
        ],
        out_specs=pl.BlockSpec(memory_space=pltpu.VMEM),
        scratch_shapes=[
            pltpu.VMEM((e_per, d, h), jnp.bfloat16),
            pltpu.VMEM((N_DEV, e_per, d, h), jnp.bfloat16),
            pltpu.SemaphoreType.DMA((N_DEV,)),
            pltpu.SemaphoreType.DMA((N_DEV,)),
            pltpu.SemaphoreType.DMA,
        ],
        compiler_params=pltpu.CompilerParams(collective_id=0),
    )(x, router_W, route_idx, expert_W)


# device time: 11071 ns/iter; 1.0044x vs baseline; 1.0044x over previous
import jax
import jax.numpy as jnp
from jax import lax
from jax.experimental import pallas as pl
from jax.experimental.pallas import tpu as pltpu

N_DEV = 4
E_PER = 2
N_EXP = N_DEV * E_PER


def kernel(x, router_W, route_idx, expert_W):
    m, d = x.shape
    e_per, _, h = expert_W.shape

    def body(x_ref, rw_ref, idx_ref, ew_ref, out_ref,
             comm_ref, send_sems, recv_sems):
        my = lax.axis_index("i")

        comm_ref[my] = ew_ref[...].astype(jnp.bfloat16)

        barrier_sem = pltpu.get_barrier_semaphore()
        for k in range(1, N_DEV):
            pl.semaphore_signal(
                barrier_sem, inc=1,
                device_id=(lax.rem(my + k, N_DEV),),
                device_id_type=pl.DeviceIdType.MESH,
            )
        pl.semaphore_wait(barrier_sem, N_DEV - 1)

        sends = []
        for k in range(1, N_DEV):
            rdma = pltpu.make_async_remote_copy(
                src_ref=comm_ref.at[my],
                dst_ref=comm_ref.at[my],
                send_sem=send_sems.at[k],
                recv_sem=recv_sems.at[my],
                device_id=(lax.rem(my + k, N_DEV),),
                device_id_type=pl.DeviceIdType.MESH,
            )
            rdma.start()
            sends.append(rdma)

        xv = x_ref[...]
        xb = xv.astype(jnp.bfloat16)
        scores = jnp.dot(xv, rw_ref[...], preferred_element_type=jnp.float32)
        p = jnp.exp(scores - jnp.max(scores, axis=-1, keepdims=True))
        p = p / jnp.sum(p, axis=-1, keepdims=True)
        iota8 = lax.broadcasted_iota(jnp.int32, (m, N_EXP), 1)
        m0 = (iota8 == idx_ref[:, 0:1]).astype(jnp.float32)
        m1 = (iota8 == idx_ref[:, 1:2]).astype(jnp.float32)
        g0 = jnp.sum(p * m0, axis=-1, keepdims=True)
        g1 = jnp.sum(p * m1, axis=-1, keepdims=True)
        gates = (g0 * m0 + g1 * m1) / (g0 + g1)

        gb = gates.astype(jnp.bfloat16)
        xs = jnp.concatenate(
            [gb[:, e:e + 1] * xb for e in range(N_EXP)], axis=1
        )

        acc = jnp.zeros((m, h), jnp.float32)
        for o in range(N_DEV):
            @pl.when(o != my)
            def _():
                recv = pltpu.make_async_remote_copy(
                    src_ref=comm_ref.at[o],
                    dst_ref=comm_ref.at[o],
                    send_sem=send_sems.at[0],
                    recv_sem=recv_sems.at[o],
                    device_id=(my,),
                    device_id_type=pl.DeviceIdType.MESH,
                )
                recv.wait_recv()
            w = jnp.reshape(comm_ref[o], (e_per * d, h))
            acc = acc + jnp.dot(
                xs[:, o * e_per * d:(o + 1) * e_per * d], w,
                preferred_element_type=jnp.float32,
            )
        out_ref[...] = acc

        for rdma in sends:
            rdma.wait_send()

    return pl.pallas_call(
        body,
        out_shape=jax.ShapeDtypeStruct((m, h), jnp.float32),
        in_specs=[
            pl.BlockSpec(memory_space=pltpu.VMEM),
            pl.BlockSpec(memory_space=pltpu.VMEM),
            pl.BlockSpec(memory_space=pltpu.VMEM),
            pl.BlockSpec(memory_space=pltpu.VMEM),
        ],
        out_specs=pl.BlockSpec(memory_space=pltpu.VMEM),
        scratch_shapes=[
            pltpu.VMEM((N_DEV, e_per, d, h), jnp.bfloat16),
            pltpu.SemaphoreType.DMA((N_DEV,)),
            pltpu.SemaphoreType.DMA((N_DEV,)),
        ],
        compiler_params=pltpu.CompilerParams(collective_id=0),
    )(x, router_W, route_idx, expert_W)
